# baseline (speedup 1.0000x reference)
import functools

import jax
import jax.numpy as jnp
from jax import lax
from jax.experimental import pallas as pl
from jax.experimental.pallas import tpu as pltpu
from jax.experimental.pallas import tpu_sc as plsc

N = 10000
E = 320000
D = 128
EPS = 1e-5

NC = 2
NS = 16
NW = NC * NS
EPT = E // NW
CHUNK = 128
NCHUNK = 80
EPT_PAD = NCHUNK * CHUNK
ACC_ROWS = 10112
DEG_ROWS = EPT_PAD // 16
NBUF = 2

_mesh = plsc.VectorSubcoreMesh(core_axis_name="c", subcore_axis_name="s")


@functools.partial(
    pl.kernel,
    out_type=jax.ShapeDtypeStruct((NW, EPT_PAD), jnp.float32),
    mesh=_mesh,
    scratch_types=[
        pltpu.VMEM((DEG_ROWS, 16), jnp.int32),
        pltpu.VMEM((EPT_PAD,), jnp.float32),
    ],
    compiler_params=pltpu.CompilerParams(needs_layout_passes=False),
)
def _deg_kernel(dst_hbm, zeros_hbm, out_hbm, dst_v, acc_v):
    c = lax.axis_index("c")
    s = lax.axis_index("s")
    wid = c * NS + s
    pltpu.sync_copy(zeros_hbm, acc_v)
    pltpu.sync_copy(dst_hbm.at[wid], dst_v)
    ones = jnp.full((16,), 1.0, jnp.float32)

    def body(j, carry):
        v = dst_v[j]
        plsc.addupdate_scatter(acc_v, [v], ones)
        return carry

    lax.fori_loop(0, DEG_ROWS, body, 0)
    pltpu.sync_copy(acc_v, out_hbm.at[wid])


@functools.partial(
    pl.kernel,
    out_type=[
        jax.ShapeDtypeStruct((ACC_ROWS, D), jnp.float32),
        jax.ShapeDtypeStruct((ACC_ROWS, D), jnp.float32),
    ],
    mesh=_mesh,
    scratch_types=[
        pltpu.VMEM((2 * NBUF, 2, CHUNK), jnp.int32),
        pltpu.VMEM((NBUF, CHUNK, D), jnp.float32),
        pltpu.VMEM_SHARED((ACC_ROWS, D), jnp.float32),
        pltpu.SemaphoreType.DMA((2 * NBUF,)),
        pltpu.SemaphoreType.DMA((NBUF,)),
        pltpu.SemaphoreType.DMA((NBUF,)),
    ],
)
def _agg_kernel(g_hbm, sd_hbm, zeros_hbm,
                out0_hbm, out1_hbm, idx_v, rows_v, acc, semi, semg, sems):
    c = lax.axis_index("c")
    s = lax.axis_index("s")
    wid = c * NS + s
    zrows = ACC_ROWS // NS
    pltpu.sync_copy(zeros_hbm.at[pl.ds(s * zrows, zrows)],
                    acc.at[pl.ds(s * zrows, zrows)])
    plsc.subcore_barrier()

    def start_idx(sl, b, jj):
        pltpu.async_copy(sd_hbm.at[wid, jj], idx_v.at[sl], semi.at[sl])

    def wait_idx(sl, b, jj):
        pltpu.make_async_copy(
            sd_hbm.at[wid, jj], idx_v.at[sl], semi.at[sl]).wait()

    def start_gather(sl, b):
        pltpu.async_copy(g_hbm.at[idx_v.at[sl, 0]], rows_v.at[b], semg.at[b])

    def wait_gather(sl, b):
        pltpu.make_async_copy(
            g_hbm.at[idx_v.at[sl, 0]], rows_v.at[b], semg.at[b]).wait()

    def start_scatter(sl, b):
        pltpu.async_copy(rows_v.at[b], acc.at[idx_v.at[sl, 1]], sems.at[b],
                         add=True)

    def wait_scatter(sl, b):
        pltpu.make_async_copy(
            rows_v.at[b], acc.at[idx_v.at[sl, 1]], sems.at[b]).wait()

    for b in range(NBUF):
        start_idx(b, b, b)
    for b in range(NBUF):
        wait_idx(b, b, b)
        start_gather(b, b)

    @pl.loop(0, NCHUNK, step=NBUF)
    def _(j):
        par = (j // NBUF) % 2
        sl0 = NBUF * par
        sl1 = NBUF - sl0
        for b in range(NBUF):
            wait_gather(sl0 + b, b)
            start_scatter(sl0 + b, b)

            @pl.when(j + b + NBUF < NCHUNK)
            def _():
                start_idx(sl1 + b, b, j + b + NBUF)
        for b in range(NBUF):
            wait_scatter(sl0 + b, b)

            @pl.when(j + b + NBUF < NCHUNK)
            def _():
                wait_idx(sl1 + b, b, j + b + NBUF)
                start_gather(sl1 + b, b)

    plsc.subcore_barrier()
    sl = pl.ds(s * zrows, zrows)

    @pl.when(c == 0)
    def _():
        pltpu.sync_copy(acc.at[sl], out0_hbm.at[sl])

    @pl.when(c == 1)
    def _():
        pltpu.sync_copy(acc.at[sl], out1_hbm.at[sl])


def _prep_body(x_ref, fcw_ref, fcb_ref, g0_ref, b0_ref, c1w_ref, degp_ref,
               h0_ref, g1_ref, dinv_ref):
    y = jnp.dot(x_ref[...], fcw_ref[...],
                preferred_element_type=jnp.float32) + fcb_ref[...]
    mu = jnp.mean(y, axis=0)
    yc = y - mu
    var = jnp.mean(yc * yc, axis=0)
    h0 = jnp.maximum(g0_ref[...] * yc * lax.rsqrt(var + EPS) + b0_ref[...], 0.0)
    h0_ref[...] = h0
    deg = jnp.sum(degp_ref[...], axis=0)[:N] + 1.0
    dinv = lax.rsqrt(deg)
    dinv_ref[...] = dinv
    g1_ref[...] = jnp.dot(h0, c1w_ref[...],
                          preferred_element_type=jnp.float32) * dinv[:, None]


def _conv_out_body(a0_ref, a1_ref, g_ref, dinv_ref, b_ref, gam_ref, bet_ref,
                   res_ref, wn_ref, out_ref, gn_ref):
    dinv = dinv_ref[...]
    z = ((a0_ref[...][:N] + a1_ref[...][:N] + g_ref[...]) * dinv[:, None]
         + b_ref[...])
    mu = jnp.mean(z, axis=0)
    zc = z - mu
    var = jnp.mean(zc * zc, axis=0)
    h = jnp.maximum(gam_ref[...] * zc * lax.rsqrt(var + EPS) + bet_ref[...],
                    0.0) + res_ref[...]
    out_ref[...] = h
    if gn_ref is not None:
        gn_ref[...] = jnp.dot(h, wn_ref[...],
                              preferred_element_type=jnp.float32) * dinv[:, None]


_prep_call = pl.pallas_call(
    _prep_body,
    out_shape=[
        jax.ShapeDtypeStruct((N, D), jnp.float32),
        jax.ShapeDtypeStruct((N, D), jnp.float32),
        jax.ShapeDtypeStruct((N,), jnp.float32),
    ],
)

_mid_call = pl.pallas_call(
    _conv_out_body,
    out_shape=[
        jax.ShapeDtypeStruct((N, D), jnp.float32),
        jax.ShapeDtypeStruct((N, D), jnp.float32),
    ],
)


def _final_body(a0, a1, g, dinv, b, gam, bet, res, out):
    _conv_out_body(a0, a1, g, dinv, b, gam, bet, res, None, out, None)


_final_call = pl.pallas_call(
    _final_body,
    out_shape=jax.ShapeDtypeStruct((N, D), jnp.float32),
)


def kernel(x, edge_index, params):
    p = params
    src = edge_index[0].reshape(NW, EPT)
    dst = edge_index[1].reshape(NW, EPT)
    padn = EPT_PAD - EPT
    pad_lanes = jnp.arange(NW * padn, dtype=jnp.int32).reshape(NW, padn)
    src_p = jnp.concatenate([src, (pad_lanes * 37) % N], axis=1)
    dst_p = jnp.concatenate(
        [dst, N + pad_lanes % (ACC_ROWS - N)], axis=1)
    src_c = src_p.reshape(NW, NCHUNK, 1, CHUNK)
    dst_c = dst_p.reshape(NW, NCHUNK, 1, CHUNK)
    sd = jnp.concatenate([src_c, dst_c], axis=2)
    dst_d = dst_p.reshape(NW, DEG_ROWS, 16)

    zeros_deg = jnp.zeros((EPT_PAD,), jnp.float32)
    zeros_acc = jnp.zeros((ACC_ROWS, D), jnp.float32)

    degp2 = _deg_kernel(dst_d, zeros_deg)

    h0, g1, dinv = _prep_call(
        x, p['fc_w'], p['fc_b'], p['bn0_g'], p['bn0_b'], p['conv1_w'], degp2)

    a0, a1 = _agg_kernel(g1, sd, zeros_acc)
    h1, g2 = _mid_call(a0, a1, g1, dinv, p['conv1_b'], p['bn1_g'], p['bn1_b'],
                       h0, p['conv2_w'])

    b0, b1 = _agg_kernel(g2, sd, zeros_acc)
    h2 = _final_call(b0, b1, g2, dinv, p['conv2_b'], p['bn2_g'], p['bn2_b'], h0)
    return h2


# speedup vs baseline: 1.1681x; 1.1681x over previous
import functools

import jax
import jax.numpy as jnp
from jax import lax
from jax.experimental import pallas as pl
from jax.experimental.pallas import tpu as pltpu
from jax.experimental.pallas import tpu_sc as plsc

N = 10000
E = 320000
D = 128
EPS = 1e-5

NC = 2
NS = 16
NW = NC * NS
EPT = E // NW
CHUNK = 64
NCHUNK = 160
EPT_PAD = NCHUNK * CHUNK
ACC_ROWS = 10112
DEG_ROWS = EPT_PAD // 16
NBUF = 5

_mesh = plsc.VectorSubcoreMesh(core_axis_name="c", subcore_axis_name="s")


@functools.partial(
    pl.kernel,
    out_type=jax.ShapeDtypeStruct((NW, EPT_PAD), jnp.float32),
    mesh=_mesh,
    scratch_types=[
        pltpu.VMEM((DEG_ROWS, 16), jnp.int32),
        pltpu.VMEM((EPT_PAD,), jnp.float32),
    ],
    compiler_params=pltpu.CompilerParams(needs_layout_passes=False),
)
def _deg_kernel(dst_hbm, zeros_hbm, out_hbm, dst_v, acc_v):
    c = lax.axis_index("c")
    s = lax.axis_index("s")
    wid = c * NS + s
    pltpu.sync_copy(zeros_hbm, acc_v)
    pltpu.sync_copy(dst_hbm.at[wid], dst_v)
    ones = jnp.full((16,), 1.0, jnp.float32)

    def body(j, carry):
        v = dst_v[j]
        plsc.addupdate_scatter(acc_v, [v], ones)
        return carry

    lax.fori_loop(0, DEG_ROWS, body, 0)
    pltpu.sync_copy(acc_v, out_hbm.at[wid])


@functools.partial(
    pl.kernel,
    out_type=[
        jax.ShapeDtypeStruct((ACC_ROWS, D), jnp.float32),
        jax.ShapeDtypeStruct((ACC_ROWS, D), jnp.float32),
    ],
    mesh=_mesh,
    scratch_types=[
        pltpu.VMEM((2 * NBUF, 2, CHUNK), jnp.int32),
        pltpu.VMEM((NBUF, CHUNK, D), jnp.float32),
        pltpu.VMEM_SHARED((ACC_ROWS, D), jnp.float32),
        pltpu.SemaphoreType.DMA((2 * NBUF,)),
        pltpu.SemaphoreType.DMA((NBUF,)),
        pltpu.SemaphoreType.DMA((NBUF,)),
    ],
)
def _agg_kernel(g_hbm, sd_hbm, zeros_hbm,
                out0_hbm, out1_hbm, idx_v, rows_v, acc, semi, semg, sems):
    c = lax.axis_index("c")
    s = lax.axis_index("s")
    wid = c * NS + s
    zrows = ACC_ROWS // NS
    pltpu.sync_copy(zeros_hbm.at[pl.ds(s * zrows, zrows)],
                    acc.at[pl.ds(s * zrows, zrows)])
    plsc.subcore_barrier()

    def start_idx(sl, b, jj):
        pltpu.async_copy(sd_hbm.at[wid, jj], idx_v.at[sl], semi.at[sl])

    def wait_idx(sl, b, jj):
        pltpu.make_async_copy(
            sd_hbm.at[wid, jj], idx_v.at[sl], semi.at[sl]).wait()

    def start_gather(sl, b):
        pltpu.async_copy(g_hbm.at[idx_v.at[sl, 0]], rows_v.at[b], semg.at[b])

    def wait_gather(sl, b):
        pltpu.make_async_copy(
            g_hbm.at[idx_v.at[sl, 0]], rows_v.at[b], semg.at[b]).wait()

    def start_scatter(sl, b):
        pltpu.async_copy(rows_v.at[b], acc.at[idx_v.at[sl, 1]], sems.at[b],
                         add=True)

    def wait_scatter(sl, b):
        pltpu.make_async_copy(
            rows_v.at[b], acc.at[idx_v.at[sl, 1]], sems.at[b]).wait()

    for b in range(NBUF):
        start_idx(b, b, b)
    for b in range(NBUF):
        wait_idx(b, b, b)
        start_gather(b, b)

    @pl.loop(0, NCHUNK, step=NBUF)
    def _(j):
        par = (j // NBUF) % 2
        sl0 = NBUF * par
        sl1 = NBUF - sl0
        for b in range(NBUF):
            wait_gather(sl0 + b, b)
            start_scatter(sl0 + b, b)

            @pl.when(j + b + NBUF < NCHUNK)
            def _():
                start_idx(sl1 + b, b, j + b + NBUF)
        for b in range(NBUF):
            wait_scatter(sl0 + b, b)

            @pl.when(j + b + NBUF < NCHUNK)
            def _():
                wait_idx(sl1 + b, b, j + b + NBUF)
                start_gather(sl1 + b, b)

    plsc.subcore_barrier()
    sl = pl.ds(s * zrows, zrows)

    @pl.when(c == 0)
    def _():
        pltpu.sync_copy(acc.at[sl], out0_hbm.at[sl])

    @pl.when(c == 1)
    def _():
        pltpu.sync_copy(acc.at[sl], out1_hbm.at[sl])


def _prep_a_body(x_ref, fcw_ref, fcb_ref, g0_ref, b0_ref, h0_ref):
    y = jnp.dot(x_ref[...], fcw_ref[...],
                preferred_element_type=jnp.float32) + fcb_ref[...]
    mu = jnp.mean(y, axis=0)
    yc = y - mu
    var = jnp.mean(yc * yc, axis=0)
    h0_ref[...] = jnp.maximum(
        g0_ref[...] * yc * lax.rsqrt(var + EPS) + b0_ref[...], 0.0)


def _prep_b_body(h0_ref, c1w_ref, degp_ref, g1_ref, dinv_ref):
    deg = jnp.sum(degp_ref[...], axis=0)[:N] + 1.0
    dinv = lax.rsqrt(deg)
    dinv_ref[...] = dinv
    g1_ref[...] = jnp.dot(h0_ref[...], c1w_ref[...],
                          preferred_element_type=jnp.float32) * dinv[:, None]


def _conv_out_body(a0_ref, a1_ref, g_ref, dinv_ref, b_ref, gam_ref, bet_ref,
                   res_ref, wn_ref, out_ref, gn_ref):
    dinv = dinv_ref[...]
    z = ((a0_ref[...][:N] + a1_ref[...][:N] + g_ref[...]) * dinv[:, None]
         + b_ref[...])
    mu = jnp.mean(z, axis=0)
    zc = z - mu
    var = jnp.mean(zc * zc, axis=0)
    h = jnp.maximum(gam_ref[...] * zc * lax.rsqrt(var + EPS) + bet_ref[...],
                    0.0) + res_ref[...]
    out_ref[...] = h
    if gn_ref is not None:
        gn_ref[...] = jnp.dot(h, wn_ref[...],
                              preferred_element_type=jnp.float32) * dinv[:, None]


_prep_a_call = pl.pallas_call(
    _prep_a_body,
    out_shape=jax.ShapeDtypeStruct((N, D), jnp.float32),
)

_prep_b_call = pl.pallas_call(
    _prep_b_body,
    out_shape=[
        jax.ShapeDtypeStruct((N, D), jnp.float32),
        jax.ShapeDtypeStruct((N,), jnp.float32),
    ],
)

_mid_call = pl.pallas_call(
    _conv_out_body,
    out_shape=[
        jax.ShapeDtypeStruct((N, D), jnp.float32),
        jax.ShapeDtypeStruct((N, D), jnp.float32),
    ],
)


def _final_body(a0, a1, g, dinv, b, gam, bet, res, out):
    _conv_out_body(a0, a1, g, dinv, b, gam, bet, res, None, out, None)


_final_call = pl.pallas_call(
    _final_body,
    out_shape=jax.ShapeDtypeStruct((N, D), jnp.float32),
)


def kernel(x, edge_index, params):
    p = params
    src = edge_index[0].reshape(NW, EPT)
    dst = edge_index[1].reshape(NW, EPT)
    padn = EPT_PAD - EPT
    pad_lanes = jnp.arange(NW * padn, dtype=jnp.int32).reshape(NW, padn)
    src_p = jnp.concatenate([src, (pad_lanes * 37) % N], axis=1)
    dst_p = jnp.concatenate(
        [dst, N + pad_lanes % (ACC_ROWS - N)], axis=1)
    src_c = src_p.reshape(NW, NCHUNK, 1, CHUNK)
    dst_c = dst_p.reshape(NW, NCHUNK, 1, CHUNK)
    sd = jnp.concatenate([src_c, dst_c], axis=2)
    dst_d = dst_p.reshape(NW, DEG_ROWS, 16)

    zeros_deg = jnp.zeros((EPT_PAD,), jnp.float32)
    zeros_acc = jnp.zeros((ACC_ROWS, D), jnp.float32)

    degp2 = _deg_kernel(dst_d, zeros_deg)
    h0 = _prep_a_call(x, p['fc_w'], p['fc_b'], p['bn0_g'], p['bn0_b'])
    g1, dinv = _prep_b_call(h0, p['conv1_w'], degp2)

    a0, a1 = _agg_kernel(g1, sd, zeros_acc)
    h1, g2 = _mid_call(a0, a1, g1, dinv, p['conv1_b'], p['bn1_g'], p['bn1_b'],
                       h0, p['conv2_w'])

    b0, b1 = _agg_kernel(g2, sd, zeros_acc)
    h2 = _final_call(b0, b1, g2, dinv, p['conv2_b'], p['bn2_g'], p['bn2_b'], h0)
    return h2


# speedup vs baseline: 1.1837x; 1.0133x over previous
import functools

import jax
import jax.numpy as jnp
from jax import lax
from jax.experimental import pallas as pl
from jax.experimental.pallas import tpu as pltpu
from jax.experimental.pallas import tpu_sc as plsc

N = 10000
E = 320000
D = 128
EPS = 1e-5

NC = 2
NS = 16
NW = NC * NS
EPT = E // NW
CHUNK = 64
NCHUNK = 160
EPT_PAD = NCHUNK * CHUNK
ACC_ROWS = 10112
DEG_ROWS = EPT_PAD // 16
NBUF = 5

_mesh = plsc.VectorSubcoreMesh(core_axis_name="c", subcore_axis_name="s")


@functools.partial(
    pl.kernel,
    out_type=jax.ShapeDtypeStruct((NW, EPT_PAD), jnp.float32),
    mesh=_mesh,
    scratch_types=[
        pltpu.VMEM((DEG_ROWS, 16), jnp.int32),
        pltpu.VMEM((EPT_PAD,), jnp.float32),
    ],
    compiler_params=pltpu.CompilerParams(needs_layout_passes=False),
)
def _deg_kernel(dst_hbm, zeros_hbm, out_hbm, dst_v, acc_v):
    c = lax.axis_index("c")
    s = lax.axis_index("s")
    wid = c * NS + s
    pltpu.sync_copy(zeros_hbm, acc_v)
    pltpu.sync_copy(dst_hbm.at[wid], dst_v)
    ones = jnp.full((16,), 1.0, jnp.float32)

    def body(j, carry):
        v = dst_v[j]
        plsc.addupdate_scatter(acc_v, [v], ones)
        return carry

    lax.fori_loop(0, DEG_ROWS, body, 0)
    pltpu.sync_copy(acc_v, out_hbm.at[wid])


@functools.partial(
    pl.kernel,
    out_type=[
        jax.ShapeDtypeStruct((ACC_ROWS, D), jnp.float32),
        jax.ShapeDtypeStruct((ACC_ROWS, D), jnp.float32),
    ],
    mesh=_mesh,
    scratch_types=[
        pltpu.VMEM((2 * NBUF, 2, CHUNK), jnp.int32),
        pltpu.VMEM((NBUF, CHUNK, D), jnp.float32),
        pltpu.VMEM_SHARED((ACC_ROWS, D), jnp.float32),
        pltpu.SemaphoreType.DMA((2 * NBUF,)),
        pltpu.SemaphoreType.DMA((NBUF,)),
        pltpu.SemaphoreType.DMA((NBUF,)),
    ],
)
def _agg_kernel(g_hbm, sd_hbm, zeros_hbm,
                out0_hbm, out1_hbm, idx_v, rows_v, acc, semi, semg, sems):
    c = lax.axis_index("c")
    s = lax.axis_index("s")
    wid = c * NS + s
    zrows = ACC_ROWS // NS
    pltpu.sync_copy(zeros_hbm.at[pl.ds(s * zrows, zrows)],
                    acc.at[pl.ds(s * zrows, zrows)])
    plsc.subcore_barrier()

    def start_idx(sl, b, jj):
        pltpu.async_copy(sd_hbm.at[wid, jj], idx_v.at[sl], semi.at[sl])

    def wait_idx(sl, b, jj):
        pltpu.make_async_copy(
            sd_hbm.at[wid, jj], idx_v.at[sl], semi.at[sl]).wait()

    def start_gather(sl, b):
        pltpu.async_copy(g_hbm.at[idx_v.at[sl, 0]], rows_v.at[b], semg.at[b])

    def wait_gather(sl, b):
        pltpu.make_async_copy(
            g_hbm.at[idx_v.at[sl, 0]], rows_v.at[b], semg.at[b]).wait()

    def start_scatter(sl, b):
        pltpu.async_copy(rows_v.at[b], acc.at[idx_v.at[sl, 1]], sems.at[b],
                         add=True)

    def wait_scatter(sl, b):
        pltpu.make_async_copy(
            rows_v.at[b], acc.at[idx_v.at[sl, 1]], sems.at[b]).wait()

    for b in range(NBUF):
        start_idx(b, b, b)
    for b in range(NBUF):
        wait_idx(b, b, b)
        start_gather(b, b)

    @pl.loop(0, NCHUNK, step=NBUF)
    def _(j):
        par = (j // NBUF) % 2
        sl0 = NBUF * par
        sl1 = NBUF - sl0
        for b in range(NBUF):
            wait_gather(sl0 + b, b)
            start_scatter(sl0 + b, b)

            @pl.when(j + b + NBUF < NCHUNK)
            def _():
                start_idx(sl1 + b, b, j + b + NBUF)
        for b in range(NBUF):
            wait_scatter(sl0 + b, b)

            @pl.when(j + b + NBUF < NCHUNK)
            def _():
                wait_idx(sl1 + b, b, j + b + NBUF)
                start_gather(sl1 + b, b)

    plsc.subcore_barrier()
    sl = pl.ds(s * zrows, zrows)

    @pl.when(c == 0)
    def _():
        pltpu.sync_copy(acc.at[sl], out0_hbm.at[sl])

    @pl.when(c == 1)
    def _():
        pltpu.sync_copy(acc.at[sl], out1_hbm.at[sl])


def _prep_body(x_ref, fcw_ref, fcb_ref, g0_ref, b0_ref, c1w_ref, degp_ref,
               h0_ref, g1_ref, dinv_ref):
    y = jnp.dot(x_ref[...], fcw_ref[...],
                preferred_element_type=jnp.float32) + fcb_ref[...]
    mu = jnp.mean(y, axis=0)
    yc = y - mu
    var = jnp.mean(yc * yc, axis=0)
    h0 = jnp.maximum(g0_ref[...] * yc * lax.rsqrt(var + EPS) + b0_ref[...], 0.0)
    h0_ref[...] = h0
    deg = jnp.sum(degp_ref[...], axis=0)[:N] + 1.0
    dinv = lax.rsqrt(deg)
    dinv_ref[...] = dinv
    g1_ref[...] = jnp.dot(h0, c1w_ref[...],
                          preferred_element_type=jnp.float32) * dinv[:, None]


def _conv_out_body(a0_ref, a1_ref, g_ref, dinv_ref, b_ref, gam_ref, bet_ref,
                   res_ref, wn_ref, out_ref, gn_ref):
    dinv = dinv_ref[...]
    z = ((a0_ref[...][:N] + a1_ref[...][:N] + g_ref[...]) * dinv[:, None]
         + b_ref[...])
    mu = jnp.mean(z, axis=0)
    zc = z - mu
    var = jnp.mean(zc * zc, axis=0)
    h = jnp.maximum(gam_ref[...] * zc * lax.rsqrt(var + EPS) + bet_ref[...],
                    0.0) + res_ref[...]
    out_ref[...] = h
    if gn_ref is not None:
        gn_ref[...] = jnp.dot(h, wn_ref[...],
                              preferred_element_type=jnp.float32) * dinv[:, None]


_prep_call = pl.pallas_call(
    _prep_body,
    out_shape=[
        jax.ShapeDtypeStruct((N, D), jnp.float32),
        jax.ShapeDtypeStruct((N, D), jnp.float32),
        jax.ShapeDtypeStruct((N,), jnp.float32),
    ],
)

_mid_call = pl.pallas_call(
    _conv_out_body,
    out_shape=[
        jax.ShapeDtypeStruct((N, D), jnp.float32),
        jax.ShapeDtypeStruct((N, D), jnp.float32),
    ],
)


def _final_body(a0, a1, g, dinv, b, gam, bet, res, out):
    _conv_out_body(a0, a1, g, dinv, b, gam, bet, res, None, out, None)


_final_call = pl.pallas_call(
    _final_body,
    out_shape=jax.ShapeDtypeStruct((N, D), jnp.float32),
)


def kernel(x, edge_index, params):
    p = params
    src = edge_index[0].reshape(NW, EPT)
    dst = edge_index[1].reshape(NW, EPT)
    padn = EPT_PAD - EPT
    pad_lanes = jnp.arange(NW * padn, dtype=jnp.int32).reshape(NW, padn)
    src_p = jnp.concatenate([src, (pad_lanes * 37) % N], axis=1)
    dst_p = jnp.concatenate(
        [dst, N + pad_lanes % (ACC_ROWS - N)], axis=1)
    src_c = src_p.reshape(NW, NCHUNK, 1, CHUNK)
    dst_c = dst_p.reshape(NW, NCHUNK, 1, CHUNK)
    sd = jnp.concatenate([src_c, dst_c], axis=2)
    dst_d = dst_p.reshape(NW, DEG_ROWS, 16)

    zeros_deg = jnp.zeros((EPT_PAD,), jnp.float32)
    zeros_acc = jnp.zeros((ACC_ROWS, D), jnp.float32)

    degp2 = _deg_kernel(dst_d, zeros_deg)
    h0, g1, dinv = _prep_call(
        x, p['fc_w'], p['fc_b'], p['bn0_g'], p['bn0_b'], p['conv1_w'], degp2)

    a0, a1 = _agg_kernel(g1, sd, zeros_acc)
    h1, g2 = _mid_call(a0, a1, g1, dinv, p['conv1_b'], p['bn1_g'], p['bn1_b'],
                       h0, p['conv2_w'])

    b0, b1 = _agg_kernel(g2, sd, zeros_acc)
    h2 = _final_call(b0, b1, g2, dinv, p['conv2_b'], p['bn2_g'], p['bn2_b'], h0)
    return h2
